# baseline (speedup 1.0000x reference)
import jax
import jax.numpy as jnp
from jax import lax
from jax.experimental import pallas as pl
from jax.experimental.pallas import tpu as pltpu
from jax.experimental.pallas import tpu_sc as plsc

N_ROWS = 10000
E = 160000
EE = 2 * E
D_IN = 256
D_OUT = 256
HALF = 128
NC = 2
NS = 16
EPT = EE // NS
CH = 80
BLK = 4000
NBLK = EPT // BLK
PAIRS = BLK // (2 * CH)
RCH = N_ROWS // CH
OCH = (RCH + NS - 1) // NS

_mesh = plsc.VectorSubcoreMesh(
    core_axis_name="c", subcore_axis_name="s", num_cores=NC, num_subcores=NS)


_BM = 1000


def _mm_body(x_ref, w_ref, o_ref):
    o_ref[0, 0] = jnp.dot(x_ref[...], w_ref[0],
                          preferred_element_type=jnp.float32)


def _tc_tables(x, wacc):
    grid = (N_ROWS // _BM, 2, 2)
    return pl.pallas_call(
        _mm_body,
        grid=grid,
        in_specs=[
            pl.BlockSpec((_BM, D_IN), lambda m, j, h: (m, 0)),
            pl.BlockSpec((1, D_IN, HALF), lambda m, j, h: (j, 0, h)),
        ],
        out_specs=pl.BlockSpec((1, 1, _BM, HALF),
                               lambda m, j, h: (h, j, m, 0)),
        out_shape=jax.ShapeDtypeStruct((2, 2, N_ROWS, HALF), jnp.float32),
    )(x, wacc)


def _scale_chunk(rows, valblk, vbase):
    @plsc.parallel_loop(0, CH // 16, 1, unroll=5)
    def grp(g):
        v16 = valblk[pl.ds(vbase + g * 16, 16)]
        for k in range(16):
            vk = lax.gather(
                v16, jnp.full((16, 1), k, jnp.int32),
                lax.GatherDimensionNumbers(
                    offset_dims=(), collapsed_slice_dims=(0,),
                    start_index_map=(0,)),
                slice_sizes=(1,),
                mode=lax.GatherScatterMode.PROMISE_IN_BOUNDS)
            r = g * 16 + k
            for j in range(HALF // 16):
                q = pl.ds(16 * j, 16)
                rows[r, q] = rows[r, q] * vk


def _sc_body(tU_ref, tI_ref, dU_ref, sU_ref, vU_ref, dI_ref, sI_ref, vI_ref,
             outU_ref, outI_ref,
             dstblk, srcblk, valblk, idxA, idxB, dstA, dstB,
             rowsA, rowsB, acc, gA, gB, sA, sB):
    c = lax.axis_index("c")
    s = lax.axis_index("s")
    coff = c * (2 * N_ROWS)

    zero16 = jnp.zeros((16,), jnp.float32)

    def zero_rowsA():
        def zrow(r, carry):
            for j in range(HALF // 16):
                rowsA[r, pl.ds(16 * j, 16)] = zero16
            return carry
        lax.fori_loop(0, CH, zrow, 0)

    def zero_acc():
        def zchunk(k, carry):
            ch = s + NS * k

            @pl.when(ch < RCH)
            def _():
                pltpu.sync_copy(rowsA, acc.at[pl.ds(ch * CH, CH)])
            return carry
        lax.fori_loop(0, OCH, zchunk, 0)

    def prep_idx(idxv, dstv, base):
        def grp(g, carry):
            q = pl.ds(g * 16, 16)
            qs = pl.ds(base + g * 16, 16)
            idxv[q] = srcblk[qs] + coff
            dstv[q] = dstblk[qs]
            return carry
        lax.fori_loop(0, CH // 16, grp, 0)

    def accumulate(t_ref, d_ref, s_ref, v_ref):
        def blk_body(b, carry):
            e0 = s * EPT + b * BLK
            qblk = pl.ds(e0, BLK)
            pltpu.sync_copy(d_ref.at[qblk], dstblk)
            pltpu.sync_copy(s_ref.at[qblk], srcblk)
            pltpu.sync_copy(v_ref.at[qblk], valblk)

            prep_idx(idxA, dstA, 0)
            pltpu.async_copy(t_ref.at[idxA], rowsA, gA)

            def pair(p, carry2):
                baseA = 2 * p * CH
                baseB = baseA + CH
                prep_idx(idxB, dstB, baseB)
                pltpu.async_copy(t_ref.at[idxB], rowsB, gB)
                pltpu.make_async_copy(t_ref.at[idxA], rowsA, gA).wait()
                _scale_chunk(rowsA, valblk, baseA)
                pltpu.async_copy(rowsA, acc.at[dstA], sA, add=True)

                @pl.when(p < PAIRS - 1)
                def _():
                    pltpu.make_async_copy(rowsA, acc.at[dstA], sA).wait()
                    prep_idx(idxA, dstA, baseA + 2 * CH)
                    pltpu.async_copy(t_ref.at[idxA], rowsA, gA)

                pltpu.make_async_copy(t_ref.at[idxB], rowsB, gB).wait()
                _scale_chunk(rowsB, valblk, baseB)
                pltpu.async_copy(rowsB, acc.at[dstB], sB, add=True)

                @pl.when(p < PAIRS - 1)
                def _():
                    pltpu.make_async_copy(rowsB, acc.at[dstB], sB).wait()
                return carry2
            lax.fori_loop(0, PAIRS, pair, 0)
            pltpu.make_async_copy(rowsA, acc.at[dstA], sA).wait()
            pltpu.make_async_copy(rowsB, acc.at[dstB], sB).wait()
            return carry
        lax.fori_loop(0, NBLK, blk_body, 0)

    def copy_out(out_ref):
        def ochunk(k, carry):
            ch = s + NS * k

            @pl.when(ch < RCH)
            def _():
                pltpu.sync_copy(acc.at[pl.ds(ch * CH, CH)], rowsA)

                def rrow(r, carry2):
                    for j in range(HALF // 16):
                        q = pl.ds(16 * j, 16)
                        rowsA[r, q] = jnp.maximum(rowsA[r, q], 0.0)
                    return carry2
                lax.fori_loop(0, CH, rrow, 0)
                pltpu.sync_copy(
                    rowsA,
                    out_ref.at[pl.ds(ch * CH, CH),
                               pl.ds(pl.multiple_of(c * HALF, HALF), HALF)])
            return carry
        lax.fori_loop(0, OCH, ochunk, 0)

    zero_rowsA()
    zero_acc()
    plsc.subcore_barrier()
    accumulate(tU_ref, dU_ref, sU_ref, vU_ref)
    plsc.subcore_barrier()
    copy_out(outU_ref)
    plsc.subcore_barrier()
    zero_rowsA()
    zero_acc()
    plsc.subcore_barrier()
    accumulate(tI_ref, dI_ref, sI_ref, vI_ref)
    plsc.subcore_barrier()
    copy_out(outI_ref)


_sc_spmm = pl.kernel(
    _sc_body,
    out_type=(jax.ShapeDtypeStruct((N_ROWS, D_OUT), jnp.float32),
              jax.ShapeDtypeStruct((N_ROWS, D_OUT), jnp.float32)),
    mesh=_mesh,
    scratch_types=[
        pltpu.VMEM((BLK,), jnp.int32),
        pltpu.VMEM((BLK,), jnp.int32),
        pltpu.VMEM((BLK,), jnp.float32),
        pltpu.VMEM((CH,), jnp.int32),
        pltpu.VMEM((CH,), jnp.int32),
        pltpu.VMEM((CH,), jnp.int32),
        pltpu.VMEM((CH,), jnp.int32),
        pltpu.VMEM((CH, HALF), jnp.float32),
        pltpu.VMEM((CH, HALF), jnp.float32),
        pltpu.VMEM_SHARED((N_ROWS, HALF), jnp.float32),
        pltpu.SemaphoreType.DMA,
        pltpu.SemaphoreType.DMA,
        pltpu.SemaphoreType.DMA,
        pltpu.SemaphoreType.DMA,
    ],
)


@jax.jit
def kernel(user_inputs, item_inputs, u_edge0, u_val0, u_edge1, u_val1,
           i_edge0, i_val0, i_edge1, i_val1, W):
    w0 = W[:, :D_OUT]
    wacc = jnp.stack([w0, w0 + W[:, D_OUT:]])
    tU = _tc_tables(item_inputs, wacc).reshape(4 * N_ROWS, HALF)
    tI = _tc_tables(user_inputs, wacc).reshape(4 * N_ROWS, HALF)

    def prep(e0, v0, e1, v1):
        e0 = e0.astype(jnp.int32)
        e1 = e1.astype(jnp.int32)
        dst = jnp.concatenate([e0[0], e1[0]])
        src = jnp.concatenate([e0[1], e1[1] + N_ROWS])
        val = jnp.concatenate([v0, v1]).astype(jnp.float32)
        return dst, src, val

    dU, sU, vU = prep(u_edge0, u_val0, u_edge1, u_val1)
    dI, sI, vI = prep(i_edge0, i_val0, i_edge1, i_val1)

    user_outputs, item_outputs = _sc_spmm(tU, tI, dU, sU, vU, dI, sI, vI)
    return (user_outputs, item_outputs)


# speedup vs baseline: 1.0112x; 1.0112x over previous
import jax
import jax.numpy as jnp
from jax import lax
from jax.experimental import pallas as pl
from jax.experimental.pallas import tpu as pltpu
from jax.experimental.pallas import tpu_sc as plsc

N_ROWS = 10000
E = 160000
EE = 2 * E
D_IN = 256
D_OUT = 256
HALF = 128
NC = 2
NS = 16
EPT = EE // NS
CH = 80
BLK = 4000
NBLK = EPT // BLK
PAIRS = BLK // (2 * CH)
RCH = N_ROWS // CH
OCH = (RCH + NS - 1) // NS

_mesh = plsc.VectorSubcoreMesh(
    core_axis_name="c", subcore_axis_name="s", num_cores=NC, num_subcores=NS)


_BM = 1000


def _mm_body(x_ref, w_ref, o_ref):
    o_ref[0, 0] = jnp.dot(x_ref[...], w_ref[0],
                          preferred_element_type=jnp.float32)


def _tc_tables(x, wacc):
    grid = (N_ROWS // _BM, 2, 2)
    return pl.pallas_call(
        _mm_body,
        grid=grid,
        in_specs=[
            pl.BlockSpec((_BM, D_IN), lambda m, j, h: (m, 0)),
            pl.BlockSpec((1, D_IN, HALF), lambda m, j, h: (j, 0, h)),
        ],
        out_specs=pl.BlockSpec((1, 1, _BM, HALF),
                               lambda m, j, h: (h, j, m, 0)),
        out_shape=jax.ShapeDtypeStruct((2, 2, N_ROWS, HALF), jnp.float32),
    )(x, wacc)


def _scale_chunk(rows, valblk, vbase):
    @plsc.parallel_loop(0, CH // 16, 1)
    def grp(g):
        v16 = valblk[pl.ds(vbase + g * 16, 16)]
        for k in range(16):
            vk = lax.gather(
                v16, jnp.full((16, 1), k, jnp.int32),
                lax.GatherDimensionNumbers(
                    offset_dims=(), collapsed_slice_dims=(0,),
                    start_index_map=(0,)),
                slice_sizes=(1,),
                mode=lax.GatherScatterMode.PROMISE_IN_BOUNDS)
            r = g * 16 + k
            for j in range(HALF // 16):
                q = pl.ds(16 * j, 16)
                rows[r, q] = rows[r, q] * vk


def _sc_body(tU_ref, tI_ref, dU_ref, sU_ref, vU_ref, dI_ref, sI_ref, vI_ref,
             outU_ref, outI_ref,
             dstblk, srcblk, valblk, idxA, idxB, dstA, dstB,
             rowsA, rowsB, acc, gA, gB, sA, sB):
    c = lax.axis_index("c")
    s = lax.axis_index("s")
    coff = c * (2 * N_ROWS)

    zero16 = jnp.zeros((16,), jnp.float32)

    def zero_rowsA():
        def zrow(r, carry):
            for j in range(HALF // 16):
                rowsA[r, pl.ds(16 * j, 16)] = zero16
            return carry
        lax.fori_loop(0, CH, zrow, 0)

    def zero_acc():
        def zchunk(k, carry):
            ch = s + NS * k

            @pl.when(ch < RCH)
            def _():
                pltpu.sync_copy(rowsA, acc.at[pl.ds(ch * CH, CH)])
            return carry
        lax.fori_loop(0, OCH, zchunk, 0)

    def prep_idx(idxv, dstv, base):
        def grp(g, carry):
            q = pl.ds(g * 16, 16)
            qs = pl.ds(base + g * 16, 16)
            idxv[q] = srcblk[qs] + coff
            dstv[q] = dstblk[qs]
            return carry
        lax.fori_loop(0, CH // 16, grp, 0)

    def accumulate(t_ref, d_ref, s_ref, v_ref):
        def blk_body(b, carry):
            e0 = s * EPT + b * BLK
            qblk = pl.ds(e0, BLK)
            pltpu.sync_copy(d_ref.at[qblk], dstblk)
            pltpu.sync_copy(s_ref.at[qblk], srcblk)
            pltpu.sync_copy(v_ref.at[qblk], valblk)

            prep_idx(idxA, dstA, 0)
            pltpu.async_copy(t_ref.at[idxA], rowsA, gA)

            def pair(p, carry2):
                baseA = 2 * p * CH
                baseB = baseA + CH
                prep_idx(idxB, dstB, baseB)
                pltpu.async_copy(t_ref.at[idxB], rowsB, gB)
                pltpu.make_async_copy(t_ref.at[idxA], rowsA, gA).wait()
                _scale_chunk(rowsA, valblk, baseA)
                pltpu.async_copy(rowsA, acc.at[dstA], sA, add=True)

                @pl.when(p < PAIRS - 1)
                def _():
                    pltpu.make_async_copy(rowsA, acc.at[dstA], sA).wait()
                    prep_idx(idxA, dstA, baseA + 2 * CH)
                    pltpu.async_copy(t_ref.at[idxA], rowsA, gA)

                pltpu.make_async_copy(t_ref.at[idxB], rowsB, gB).wait()
                _scale_chunk(rowsB, valblk, baseB)
                pltpu.async_copy(rowsB, acc.at[dstB], sB, add=True)

                @pl.when(p < PAIRS - 1)
                def _():
                    pltpu.make_async_copy(rowsB, acc.at[dstB], sB).wait()
                return carry2
            lax.fori_loop(0, PAIRS, pair, 0)
            pltpu.make_async_copy(rowsA, acc.at[dstA], sA).wait()
            pltpu.make_async_copy(rowsB, acc.at[dstB], sB).wait()
            return carry
        lax.fori_loop(0, NBLK, blk_body, 0)

    def copy_out(out_ref):
        def ochunk(k, carry):
            ch = s + NS * k

            @pl.when(ch < RCH)
            def _():
                pltpu.sync_copy(acc.at[pl.ds(ch * CH, CH)], rowsA)

                def rrow(r, carry2):
                    for j in range(HALF // 16):
                        q = pl.ds(16 * j, 16)
                        rowsA[r, q] = jnp.maximum(rowsA[r, q], 0.0)
                    return carry2
                lax.fori_loop(0, CH, rrow, 0)
                pltpu.sync_copy(
                    rowsA,
                    out_ref.at[pl.ds(ch * CH, CH),
                               pl.ds(pl.multiple_of(c * HALF, HALF), HALF)])
            return carry
        lax.fori_loop(0, OCH, ochunk, 0)

    zero_rowsA()
    zero_acc()
    plsc.subcore_barrier()
    accumulate(tU_ref, dU_ref, sU_ref, vU_ref)
    plsc.subcore_barrier()
    copy_out(outU_ref)
    plsc.subcore_barrier()
    zero_rowsA()
    zero_acc()
    plsc.subcore_barrier()
    accumulate(tI_ref, dI_ref, sI_ref, vI_ref)
    plsc.subcore_barrier()
    copy_out(outI_ref)


_sc_spmm = pl.kernel(
    _sc_body,
    out_type=(jax.ShapeDtypeStruct((N_ROWS, D_OUT), jnp.float32),
              jax.ShapeDtypeStruct((N_ROWS, D_OUT), jnp.float32)),
    mesh=_mesh,
    scratch_types=[
        pltpu.VMEM((BLK,), jnp.int32),
        pltpu.VMEM((BLK,), jnp.int32),
        pltpu.VMEM((BLK,), jnp.float32),
        pltpu.VMEM((CH,), jnp.int32),
        pltpu.VMEM((CH,), jnp.int32),
        pltpu.VMEM((CH,), jnp.int32),
        pltpu.VMEM((CH,), jnp.int32),
        pltpu.VMEM((CH, HALF), jnp.float32),
        pltpu.VMEM((CH, HALF), jnp.float32),
        pltpu.VMEM_SHARED((N_ROWS, HALF), jnp.float32),
        pltpu.SemaphoreType.DMA,
        pltpu.SemaphoreType.DMA,
        pltpu.SemaphoreType.DMA,
        pltpu.SemaphoreType.DMA,
    ],
)


@jax.jit
def kernel(user_inputs, item_inputs, u_edge0, u_val0, u_edge1, u_val1,
           i_edge0, i_val0, i_edge1, i_val1, W):
    w0 = W[:, :D_OUT]
    wacc = jnp.stack([w0, w0 + W[:, D_OUT:]])
    tU = _tc_tables(item_inputs, wacc).reshape(4 * N_ROWS, HALF)
    tI = _tc_tables(user_inputs, wacc).reshape(4 * N_ROWS, HALF)

    def prep(e0, v0, e1, v1):
        e0 = e0.astype(jnp.int32)
        e1 = e1.astype(jnp.int32)
        dst = jnp.concatenate([e0[0], e1[0]])
        src = jnp.concatenate([e0[1], e1[1] + N_ROWS])
        val = jnp.concatenate([v0, v1]).astype(jnp.float32)
        return dst, src, val

    dU, sU, vU = prep(u_edge0, u_val0, u_edge1, u_val1)
    dI, sI, vI = prep(i_edge0, i_val0, i_edge1, i_val1)

    user_outputs, item_outputs = _sc_spmm(tU, tI, dU, sU, vU, dI, sI, vI)
    return (user_outputs, item_outputs)
